# baseline (speedup 1.0000x reference)
import functools

import jax
import jax.numpy as jnp
from jax import lax
from jax.experimental import pallas as pl
from jax.experimental.pallas import tpu as pltpu

N = 50000
F_IN = 16
E = 40000
S = 10000
EO = 20000
G = 256
EA = 5

_TE = 400
_TN = 400


def _elu(v):
    return jnp.where(v > 0.0, v, jnp.exp(jnp.minimum(v, 0.0)) - 1.0)


def _edge_msg_body(i_dim, o_dim, xj_ref, ea_ref, w1_ref, b1_ref, w2_ref,
                   b2_ref, out_ref):
    h = jnp.maximum(
        jnp.dot(ea_ref[...], w1_ref[...], preferred_element_type=jnp.float32)
        + b1_ref[...], 0.0)
    wt = jnp.dot(h, w2_ref[...], preferred_element_type=jnp.float32) + b2_ref[...]
    xj = xj_ref[...]
    acc = xj[:, 0:1] * wt[:, 0:o_dim]
    for ii in range(1, i_dim):
        acc = acc + xj[:, ii:ii + 1] * wt[:, ii * o_dim:(ii + 1) * o_dim]
    out_ref[...] = acc


def _edge_messages(xj, ea, w1, b1, w2, b2, i_dim, o_dim):
    e = xj.shape[0]
    grid = e // _TE
    return pl.pallas_call(
        functools.partial(_edge_msg_body, i_dim, o_dim),
        grid=(grid,),
        in_specs=[
            pl.BlockSpec((_TE, i_dim), lambda g: (g, 0)),
            pl.BlockSpec((_TE, EA), lambda g: (g, 0)),
            pl.BlockSpec((EA, 128), lambda g: (0, 0)),
            pl.BlockSpec((1, 128), lambda g: (0, 0)),
            pl.BlockSpec((128, i_dim * o_dim), lambda g: (0, 0)),
            pl.BlockSpec((1, i_dim * o_dim), lambda g: (0, 0)),
        ],
        out_specs=pl.BlockSpec((_TE, o_dim), lambda g: (g, 0)),
        out_shape=jax.ShapeDtypeStruct((e, o_dim), jnp.float32),
    )(xj, ea, w1, b1.reshape(1, -1), w2, b2.reshape(1, -1))


def _node_body(agg_ref, x_ref, root_ref, bias_ref, out_ref):
    v = (agg_ref[...]
         + jnp.dot(x_ref[...], root_ref[...], preferred_element_type=jnp.float32)
         + bias_ref[...])
    out_ref[...] = _elu(v)


def _node_update(agg, x, root, bias):
    n, i_dim = x.shape
    o_dim = root.shape[1]
    grid = n // _TN
    return pl.pallas_call(
        _node_body,
        grid=(grid,),
        in_specs=[
            pl.BlockSpec((_TN, o_dim), lambda g: (g, 0)),
            pl.BlockSpec((_TN, i_dim), lambda g: (g, 0)),
            pl.BlockSpec((i_dim, o_dim), lambda g: (0, 0)),
            pl.BlockSpec((1, o_dim), lambda g: (0, 0)),
        ],
        out_specs=pl.BlockSpec((_TN, o_dim), lambda g: (g, 0)),
        out_shape=jax.ShapeDtypeStruct((n, o_dim), jnp.float32),
    )(agg, x, root, bias.reshape(1, -1))


def _fc_body(x_ref, w1_ref, b1_ref, w2_ref, b2_ref, w3_ref, b3_ref, out_ref):
    v = _elu(jnp.dot(x_ref[...], w1_ref[...], preferred_element_type=jnp.float32)
             + b1_ref[...])
    v = _elu(jnp.dot(v, w2_ref[...], preferred_element_type=jnp.float32)
             + b2_ref[...])
    out_ref[...] = (jnp.dot(v, w3_ref[...], preferred_element_type=jnp.float32)
                    + b3_ref[...])


def _fc_stack(x, w1, b1, w2, b2, w3, b3):
    return pl.pallas_call(
        _fc_body,
        out_shape=jax.ShapeDtypeStruct((G, 1), jnp.float32),
    )(x, w1, b1.reshape(1, -1), w2, b2.reshape(1, -1), w3, b3.reshape(1, -1))


def _nnconv_layer(x, src, dst, ea, w1, b1, w2, b2, root, bias):
    i_dim, o_dim = root.shape
    xj = jnp.take(x, src, axis=0)
    msg = _edge_messages(xj, ea, w1, b1, w2, b2, i_dim, o_dim)
    agg = jax.ops.segment_sum(msg, dst, num_segments=x.shape[0])
    return _node_update(agg, x, root, bias)


def _mean_pool(x, seg, n_out):
    s = jax.ops.segment_sum(x, seg, num_segments=n_out)
    c = jax.ops.segment_sum(jnp.ones((x.shape[0], 1), x.dtype), seg,
                            num_segments=n_out)
    return s / jnp.maximum(c, 1.0)


def kernel(x, edge_index, edge_attr, node_to_subgraph, original_edge_index, original_edge_attr, subgraph_to_graph, sub0_nw1, sub0_nb1, sub0_nw2, sub0_nb2, sub0_root, sub0_bias, sub1_nw1, sub1_nb1, sub1_nw2, sub1_nb2, sub1_root, sub1_bias, sub2_nw1, sub2_nb1, sub2_nw2, sub2_nb2, sub2_root, sub2_bias, gl0_nw1, gl0_nb1, gl0_nw2, gl0_nb2, gl0_root, gl0_bias, gl1_nw1, gl1_nb1, gl1_nw2, gl1_nb2, gl1_root, gl1_bias, fc1_w, fc1_b, fc2_w, fc2_b, fc3_w, fc3_b):
    src, dst = edge_index[0], edge_index[1]
    x = _nnconv_layer(x, src, dst, edge_attr,
                      sub0_nw1, sub0_nb1, sub0_nw2, sub0_nb2, sub0_root, sub0_bias)
    x = _nnconv_layer(x, src, dst, edge_attr,
                      sub1_nw1, sub1_nb1, sub1_nw2, sub1_nb2, sub1_root, sub1_bias)
    x = _nnconv_layer(x, src, dst, edge_attr,
                      sub2_nw1, sub2_nb1, sub2_nw2, sub2_nb2, sub2_root, sub2_bias)
    x = _mean_pool(x, node_to_subgraph, S)
    osrc, odst = original_edge_index[0], original_edge_index[1]
    x = _nnconv_layer(x, osrc, odst, original_edge_attr,
                      gl0_nw1, gl0_nb1, gl0_nw2, gl0_nb2, gl0_root, gl0_bias)
    x = _nnconv_layer(x, osrc, odst, original_edge_attr,
                      gl1_nw1, gl1_nb1, gl1_nw2, gl1_nb2, gl1_root, gl1_bias)
    x = _mean_pool(x, subgraph_to_graph, G)
    return _fc_stack(x, fc1_w, fc1_b, fc2_w, fc2_b, fc3_w, fc3_b).reshape(-1)
